# baseline (speedup 1.0000x reference)
import functools

import jax
import jax.numpy as jnp
from jax import lax
from jax.experimental import pallas as pl
from jax.experimental.pallas import tpu as pltpu
from jax.experimental.pallas import tpu_sc as plsc

_ROWS = 128
_EMB = 49152
_BATCH = 8
_NUM_WORKERS = 32
_ROWS_PER_WORKER = _ROWS // _NUM_WORKERS

_mesh = plsc.VectorSubcoreMesh(core_axis_name="c", subcore_axis_name="s")


@functools.partial(
    pl.kernel,
    out_type=jax.ShapeDtypeStruct((_BATCH, _ROWS, _EMB), jnp.float32),
    mesh=_mesh,
    scratch_types=[
        pltpu.VMEM((4, _EMB // 2), jnp.float32),
        pltpu.SemaphoreType.DMA,
        pltpu.SemaphoreType.DMA,
    ],
)
def _broadcast_table(table_hbm, out_hbm, buf, in_sem, out_sem):
    wid = lax.axis_index("s") * 2 + lax.axis_index("c")
    base = wid * _ROWS_PER_WORKER
    half = _EMB // 2
    n_chunks = _ROWS_PER_WORKER * 2

    def chunk_src(k):
        row, h = base + k // 2, (k % 2) * half
        return table_hbm.at[row, pl.ds(h, half)]

    reads = [None] * n_chunks
    writes = [[] for _ in range(n_chunks)]
    for k in range(2):
        reads[k] = pltpu.async_copy(chunk_src(k), buf.at[k], in_sem)
    for k in range(n_chunks):
        if k >= 2:
            for w in writes[k - 2]:
                w.wait()
        if k + 2 < n_chunks:
            reads[k + 2] = pltpu.async_copy(
                chunk_src(k + 2), buf.at[(k + 2) % 4], in_sem
            )
        reads[k].wait()
        row, h = base + k // 2, (k % 2) * half
        writes[k] = [
            pltpu.async_copy(
                buf.at[k % 4],
                out_hbm.at[b, row, pl.ds(h, half)],
                out_sem,
            )
            for b in range(_BATCH)
        ]
    for k in (n_chunks - 2, n_chunks - 1):
        for w in writes[k]:
            w.wait()


def kernel(batch_size, table):
    del batch_size
    return _broadcast_table(table)


# speedup vs baseline: 1.0158x; 1.0158x over previous
import functools

import jax
import jax.numpy as jnp
from jax import lax
from jax.experimental import pallas as pl
from jax.experimental.pallas import tpu as pltpu
from jax.experimental.pallas import tpu_sc as plsc

_ROWS = 128
_EMB = 49152
_BATCH = 8
_NUM_WORKERS = 32
_ROWS_PER_WORKER = _ROWS // _NUM_WORKERS

_mesh = plsc.VectorSubcoreMesh(core_axis_name="c", subcore_axis_name="s")


@functools.partial(
    pl.kernel,
    out_type=jax.ShapeDtypeStruct((_BATCH, _ROWS, _EMB), jnp.float32),
    mesh=_mesh,
    scratch_types=[
        pltpu.VMEM((2, _EMB), jnp.float32),
        pltpu.SemaphoreType.DMA,
        pltpu.SemaphoreType.DMA,
    ],
)
def _broadcast_table(table_hbm, out_hbm, buf, in_sem, out_sem):
    wid = lax.axis_index("s") * 2 + lax.axis_index("c")
    base = wid * _ROWS_PER_WORKER

    read = pltpu.async_copy(
        table_hbm.at[pl.ds(base, 1)], buf.at[pl.ds(0, 1)], in_sem
    )
    pending_writes = []
    for r in range(_ROWS_PER_WORKER):
        slot = r % 2
        for w in pending_writes:
            w.wait()
        next_read = None
        if r + 1 < _ROWS_PER_WORKER:
            next_read = pltpu.async_copy(
                table_hbm.at[pl.ds(base + r + 1, 1)],
                buf.at[pl.ds(1 - slot, 1)],
                in_sem,
            )
        read.wait()
        pending_writes = [
            pltpu.async_copy(
                buf.at[pl.ds(slot, 1)],
                out_hbm.at[b].at[pl.ds(base + r, 1)],
                out_sem,
            )
            for b in range(_BATCH)
        ]
        read = next_read
    for w in pending_writes:
        w.wait()


def kernel(batch_size, table):
    del batch_size
    return _broadcast_table(table)
